# baseline (speedup 1.0000x reference)
import jax
import jax.numpy as jnp
from jax import lax
from jax.experimental import pallas as pl

_B, _T, _D = 16, 2048, 512
_N_CB, _CB_SIZE, _CB_DIM = 9, 1024, 8
_TOK = _B * _T
_BLK = 1024
_GRID = _TOK // _BLK
_EPS = 1e-12
_PREC = lax.Precision.DEFAULT
_PREC_EXACT = lax.Precision.HIGHEST


def _rvq_kernel(z_ref, inv_ref, ing_ref, inb_ref, outv_ref, outg_ref,
                outb_ref, cb_ref, cbt_ref,
                zq_ref, codes_ref, lat_ref, loss_ref):
    pid = pl.program_id(0)

    @pl.when(pid == 0)
    def _init_loss():
        loss_ref[...] = jnp.zeros((8, 128), jnp.float32)

    res = z_ref[...]
    inv = inv_ref[...]
    ing = ing_ref[0:1, :]
    inb = inb_ref[0:1, :]
    outv = outv_ref[...]
    cbt = cbt_ref[...]

    in_nrm = jnp.sqrt(jnp.sum(inv * inv, axis=0, keepdims=True))
    win = (ing * inv) / in_nrm

    zq = jnp.zeros((_BLK, _D), jnp.float32)
    loss = jnp.zeros((), jnp.float32)
    lats = []
    codes = []
    lane_iota = lax.broadcasted_iota(jnp.int32, (_BLK, _CB_SIZE), 1)

    for i in range(_N_CB):
        sl = slice(8 * i, 8 * (i + 1))
        z_e = jnp.dot(res, win[:, sl], precision=_PREC) + inb[:, sl]
        enc_nrm = jnp.sqrt(jnp.sum(z_e * z_e, axis=1, keepdims=True))
        enc_n = z_e / jnp.maximum(_EPS, enc_nrm)
        r2 = jnp.sum(enc_n * enc_n, axis=1, keepdims=True)
        cbt_i = cbt[sl, :]
        cb_nrm = jnp.sqrt(jnp.sum(cbt_i * cbt_i, axis=0, keepdims=True))
        cbt_n = cbt_i / jnp.maximum(_EPS, cb_nrm)
        c2 = jnp.sum(cbt_n * cbt_n, axis=0, keepdims=True)
        s = jnp.dot(enc_n, cbt_n, precision=_PREC)
        neg = -(r2 - 2.0 * s + c2)
        mx = jnp.max(neg, axis=1, keepdims=True)
        idx = jnp.min(jnp.where(neg == mx, lane_iota, _CB_SIZE),
                      axis=1).astype(jnp.int32)
        onehot = (lane_iota == idx[:, None]).astype(jnp.float32)
        cb_i = cb_ref[1024 * i:1024 * (i + 1), :]
        z_q_lat = jnp.dot(onehot, cb_i, precision=_PREC_EXACT)
        diff = z_e - z_q_lat
        loss = loss + jnp.sum(diff * diff)
        outv_i = outv[sl, :]
        out_nrm = jnp.sqrt(jnp.sum(outv_i * outv_i, axis=0, keepdims=True))
        wout = (outg_ref[i:i + 1, :] * outv_i) / out_nrm
        z_q_i = jnp.dot(z_q_lat, wout, precision=_PREC) + outb_ref[i:i + 1, :]
        zq = zq + z_q_i
        res = res - z_q_i
        lats.append(z_e)
        codes.append(idx)

    zq_ref[...] = zq
    lat_ref[...] = jnp.concatenate(lats, axis=1)
    codes_ref[...] = jnp.stack(codes, axis=1)
    loss_ref[...] += jnp.full((8, 128), loss, jnp.float32)


def kernel(z, in_v, in_g, in_b, out_v, out_g, out_b, codebooks):
    zf = z.reshape(_TOK, _D)
    inv_cat = in_v.transpose(1, 0, 2).reshape(_D, _N_CB * _CB_DIM)
    ing = jnp.pad(in_g.reshape(1, -1), ((0, 7), (0, 0)))
    inb = jnp.pad(in_b.reshape(1, -1), ((0, 7), (0, 0)))
    outv_cat = out_v.reshape(_N_CB * _CB_DIM, _D)
    outg = jnp.pad(out_g, ((0, 7), (0, 0)))
    outb = jnp.pad(out_b, ((0, 7), (0, 0)))
    cb_cat = codebooks.reshape(_N_CB * _CB_SIZE, _CB_DIM)
    cbt_cat = codebooks.transpose(0, 2, 1).reshape(_N_CB * _CB_DIM, _CB_SIZE)

    full = lambda shape: pl.BlockSpec(shape, lambda i: (0,) * len(shape))
    zq_f, codes_f, lat_f, loss_arr = pl.pallas_call(
        _rvq_kernel,
        grid=(_GRID,),
        in_specs=[
            pl.BlockSpec((_BLK, _D), lambda i: (i, 0)),
            full((_D, _N_CB * _CB_DIM)),
            full((8, _N_CB * _CB_DIM)),
            full((8, _N_CB * _CB_DIM)),
            full((_N_CB * _CB_DIM, _D)),
            full((16, _D)),
            full((16, _D)),
            full((_N_CB * _CB_SIZE, _CB_DIM)),
            full((_N_CB * _CB_DIM, _CB_SIZE)),
        ],
        out_specs=[
            pl.BlockSpec((_BLK, _D), lambda i: (i, 0)),
            pl.BlockSpec((_BLK, _N_CB), lambda i: (i, 0)),
            pl.BlockSpec((_BLK, _N_CB * _CB_DIM), lambda i: (i, 0)),
            full((8, 128)),
        ],
        out_shape=[
            jax.ShapeDtypeStruct((_TOK, _D), jnp.float32),
            jax.ShapeDtypeStruct((_TOK, _N_CB), jnp.int32),
            jax.ShapeDtypeStruct((_TOK, _N_CB * _CB_DIM), jnp.float32),
            jax.ShapeDtypeStruct((8, 128), jnp.float32),
        ],
    )(zf, inv_cat, ing, inb, outv_cat, outg, outb, cb_cat, cbt_cat)

    z_q = zq_f.reshape(_B, _T, _D)
    codes = codes_f.reshape(_B, _T, _N_CB)
    latents = lat_f.reshape(_B, _T, _N_CB * _CB_DIM)
    loss = loss_arr[0, 0] / jnp.float32(_B * _T * _CB_DIM)
    return (z_q, codes, latents, loss, loss)


# speedup vs baseline: 3.0082x; 3.0082x over previous
import jax
import jax.numpy as jnp
from jax import lax
from jax.experimental import pallas as pl

_B, _T, _D = 16, 2048, 512
_N_CB, _CB_SIZE, _CB_DIM = 9, 1024, 8
_TOK = _B * _T
_BLK = 1024
_GRID = _TOK // _BLK
_EPS = 1e-12
_PREC = lax.Precision.DEFAULT
_PREC_EXACT = lax.Precision.HIGHEST


def _rvq_kernel(z_ref, inv_ref, ing_ref, inb_ref, outv_ref, outg_ref,
                outb_ref, cb_ref, cbt_ref,
                zq_ref, codes_ref, lat_ref, loss_ref):
    pid = pl.program_id(0)

    @pl.when(pid == 0)
    def _init_loss():
        loss_ref[...] = jnp.zeros((8, 128), jnp.float32)

    z_in = z_ref[...]
    res = z_in
    inv = inv_ref[...]
    ing = ing_ref[0:1, :]
    inb = inb_ref[0:1, :]
    outv = outv_ref[...]
    cbt = cbt_ref[...]

    in_nrm = jnp.sqrt(jnp.sum(inv * inv, axis=0, keepdims=True))
    win = (ing * inv) / in_nrm

    loss = jnp.zeros((), jnp.float32)
    lats = []
    codes = []
    lane_iota = lax.broadcasted_iota(jnp.int32, (_BLK, _CB_SIZE), 1)

    for i in range(_N_CB):
        sl = slice(8 * i, 8 * (i + 1))
        z_e = jnp.dot(res, win[:, sl], precision=_PREC) + inb[:, sl]
        enc_nrm = jnp.sqrt(jnp.sum(z_e * z_e, axis=1, keepdims=True))
        enc_n = z_e / jnp.maximum(_EPS, enc_nrm)
        cbt_i = cbt[sl, :]
        cb_nrm = jnp.sqrt(jnp.sum(cbt_i * cbt_i, axis=0, keepdims=True))
        cbt_n = cbt_i / jnp.maximum(_EPS, cb_nrm)
        c2 = jnp.sum(cbt_n * cbt_n, axis=0, keepdims=True)
        s2 = jnp.dot(enc_n, cbt_n * 2.0, precision=_PREC)
        score = s2 - c2
        mx = jnp.max(score, axis=1, keepdims=True)
        idx = jnp.min(jnp.where(score == mx, lane_iota, _CB_SIZE),
                      axis=1).astype(jnp.int32)
        onehot = (lane_iota == idx[:, None]).astype(jnp.bfloat16)
        cb_i = cb_ref[1024 * i:1024 * (i + 1), :]
        z_q_lat = jnp.dot(onehot, cb_i.astype(jnp.bfloat16),
                          preferred_element_type=jnp.float32)
        diff = z_e - z_q_lat
        loss = loss + jnp.sum(diff * diff)
        outv_i = outv[sl, :]
        out_nrm = jnp.sqrt(jnp.sum(outv_i * outv_i, axis=0, keepdims=True))
        wout = (outg_ref[i:i + 1, :] * outv_i) / out_nrm
        z_q_i = jnp.dot(z_q_lat, wout, precision=_PREC) + outb_ref[i:i + 1, :]
        res = res - z_q_i
        lats.append(z_e)
        codes.append(idx)

    zq_ref[...] = z_in - res
    lat_ref[...] = jnp.concatenate(lats, axis=1)
    codes_ref[...] = jnp.stack(codes, axis=1)
    loss_ref[...] += jnp.full((8, 128), loss, jnp.float32)


def kernel(z, in_v, in_g, in_b, out_v, out_g, out_b, codebooks):
    zf = z.reshape(_TOK, _D)
    inv_cat = in_v.transpose(1, 0, 2).reshape(_D, _N_CB * _CB_DIM)
    ing = jnp.pad(in_g.reshape(1, -1), ((0, 7), (0, 0)))
    inb = jnp.pad(in_b.reshape(1, -1), ((0, 7), (0, 0)))
    outv_cat = out_v.reshape(_N_CB * _CB_DIM, _D)
    outg = jnp.pad(out_g, ((0, 7), (0, 0)))
    outb = jnp.pad(out_b, ((0, 7), (0, 0)))
    cb_cat = codebooks.reshape(_N_CB * _CB_SIZE, _CB_DIM)
    cbt_cat = codebooks.transpose(0, 2, 1).reshape(_N_CB * _CB_DIM, _CB_SIZE)

    full = lambda shape: pl.BlockSpec(shape, lambda i: (0,) * len(shape))
    zq_f, codes_f, lat_f, loss_arr = pl.pallas_call(
        _rvq_kernel,
        grid=(_GRID,),
        in_specs=[
            pl.BlockSpec((_BLK, _D), lambda i: (i, 0)),
            full((_D, _N_CB * _CB_DIM)),
            full((8, _N_CB * _CB_DIM)),
            full((8, _N_CB * _CB_DIM)),
            full((_N_CB * _CB_DIM, _D)),
            full((16, _D)),
            full((16, _D)),
            full((_N_CB * _CB_SIZE, _CB_DIM)),
            full((_N_CB * _CB_DIM, _CB_SIZE)),
        ],
        out_specs=[
            pl.BlockSpec((_BLK, _D), lambda i: (i, 0)),
            pl.BlockSpec((_BLK, _N_CB), lambda i: (i, 0)),
            pl.BlockSpec((_BLK, _N_CB * _CB_DIM), lambda i: (i, 0)),
            full((8, 128)),
        ],
        out_shape=[
            jax.ShapeDtypeStruct((_TOK, _D), jnp.float32),
            jax.ShapeDtypeStruct((_TOK, _N_CB), jnp.int32),
            jax.ShapeDtypeStruct((_TOK, _N_CB * _CB_DIM), jnp.float32),
            jax.ShapeDtypeStruct((8, 128), jnp.float32),
        ],
    )(zf, inv_cat, ing, inb, outv_cat, outg, outb, cb_cat, cbt_cat)

    z_q = zq_f.reshape(_B, _T, _D)
    codes = codes_f.reshape(_B, _T, _N_CB)
    latents = lat_f.reshape(_B, _T, _N_CB * _CB_DIM)
    loss = loss_arr[0, 0] / jnp.float32(_B * _T * _CB_DIM)
    return (z_q, codes, latents, loss, loss)
